# baseline (speedup 1.0000x reference)
import functools

import jax
import jax.numpy as jnp
from jax import lax
from jax.experimental import pallas as pl
from jax.experimental.pallas import tpu as pltpu
from jax.experimental.pallas import tpu_sc as plsc

NUM_LAYERS = 3
ENC_DIM = 64
N_CAT = NUM_LAYERS + 2
N_PAIR = N_CAT * N_CAT
BATCH = 4096
NEIGH = 50
NPAD = 56

NC, NS, LANES = 2, 16, 16
NW = NC * NS
BBLK = BATCH // NW
CODE_PITCH = 128


def _sc_body(node_hbm, t_hbm, emb_hbm, out_hbm,
             emb_v, stT, n_v, t_v, stage, sem_out0, sem_out1):
    cid = lax.axis_index("c")
    sid = lax.axis_index("s")
    wid = sid * NC + cid
    wb = pl.multiple_of(wid * BBLK, BBLK)
    sems = (sem_out0, sem_out1)

    pltpu.sync_copy(emb_hbm, emb_v)
    iota = lax.iota(jnp.int32, LANES)
    for dc in range(ENC_DIM // LANES):
        dsl = pl.ds(dc * LANES, LANES)
        base16 = (iota + dc * LANES) * CODE_PITCH
        for i in range(N_CAT):
            for j in range(N_CAT):
                val = emb_v[i, dsl] + emb_v[j, dsl]
                plsc.store_scatter(stT, [base16 + (i * N_CAT + j)], val)

    pltpu.sync_copy(node_hbm.at[:, pl.ds(wb, BBLK)], n_v)
    pltpu.sync_copy(t_hbm.at[:, pl.ds(wb, BBLK)], t_v)

    def emit_n(n, p):
        for bc in range(BBLK // LANES):
            sl = pl.ds(bc * LANES, LANES)
            nn = n_v[n, sl]
            tt = t_v[n, sl]
            s = nn + (tt * 1000.0).astype(jnp.int32)
            d1 = nn - N_CAT * ((nn * 26215) >> 17)
            d2 = s - N_CAT * ((s * 26215) >> 17)
            code = jnp.where(nn == 0, N_PAIR - 1, d1 * N_CAT + d2)

            @plsc.parallel_loop(0, ENC_DIM, unroll=16)
            def _(d, code=code, sl=sl, p=p):
                stage[p, d, sl] = plsc.load_gather(stT, [code + d * CODE_PITCH])
        pltpu.async_copy(stage.at[p], out_hbm.at[n, :, pl.ds(wb, BBLK)],
                         sems[p])

    def loop_body(i, carry):
        for p in range(2):
            @pl.when(i >= 1)
            def _(p=p):
                pltpu.make_async_copy(
                    stage.at[p], out_hbm.at[0, :, pl.ds(wb, BBLK)], sems[p]
                ).wait()
            emit_n(2 * i + p, p)
        return carry

    lax.fori_loop(0, NEIGH // 2, loop_body, 0)
    for p in range(2):
        pltpu.make_async_copy(
            stage.at[p], out_hbm.at[0, :, pl.ds(wb, BBLK)], sems[p]
        ).wait()


@functools.partial(
    pl.kernel,
    out_type=jax.ShapeDtypeStruct((NEIGH, ENC_DIM, BATCH), jnp.float32),
    mesh=plsc.VectorSubcoreMesh(core_axis_name="c", subcore_axis_name="s",
                                num_cores=NC, num_subcores=NS),
    compiler_params=pltpu.CompilerParams(use_tc_tiling_on_sc=True,
                                         needs_layout_passes=False),
    scratch_types=[
        pltpu.VMEM((N_CAT, ENC_DIM), jnp.float32),
        pltpu.VMEM((ENC_DIM * CODE_PITCH,), jnp.float32),
        pltpu.VMEM((NPAD, BBLK), jnp.int32),
        pltpu.VMEM((NPAD, BBLK), jnp.float32),
        pltpu.VMEM((2, ENC_DIM, BBLK), jnp.float32),
        pltpu.SemaphoreType.DMA,
        pltpu.SemaphoreType.DMA,
    ],
)
def _sc_encode(node_hbm, t_hbm, emb_hbm, out_hbm,
               emb_v, stT, n_v, t_v, stage, sem_out0, sem_out1):
    _sc_body(node_hbm, t_hbm, emb_hbm, out_hbm,
             emb_v, stT, n_v, t_v, stage, sem_out0, sem_out1)


def kernel(node_record, t_record, emb_table):
    nodeT = jnp.pad(node_record.transpose(1, 0), ((0, NPAD - NEIGH), (0, 0)))
    tT = jnp.pad(t_record.transpose(1, 0), ((0, NPAD - NEIGH), (0, 0)))
    outT = _sc_encode(nodeT, tT, emb_table)
    return outT.transpose(2, 0, 1)


# speedup vs baseline: 1.0644x; 1.0644x over previous
import functools

import jax
import jax.numpy as jnp
from jax import lax
from jax.experimental import pallas as pl
from jax.experimental.pallas import tpu as pltpu
from jax.experimental.pallas import tpu_sc as plsc

NUM_LAYERS = 3
ENC_DIM = 64
N_CAT = NUM_LAYERS + 2
N_PAIR = N_CAT * N_CAT
BATCH = 4096
NEIGH = 50
NPAD = 56

NC, NS, LANES = 2, 16, 16
NW = NC * NS
BBLK = BATCH // NW
CODE_PITCH = 128


def _sc_body(node_hbm, t_hbm, emb_hbm, out_hbm,
             emb_v, stT, n_v, t_v, stage, sem_out0, sem_out1):
    cid = lax.axis_index("c")
    sid = lax.axis_index("s")
    wid = sid * NC + cid
    wb = pl.multiple_of(wid * BBLK, BBLK)
    sems = (sem_out0, sem_out1)

    pltpu.sync_copy(emb_hbm, emb_v)
    iota = lax.iota(jnp.int32, LANES)
    for dc in range(ENC_DIM // LANES):
        dsl = pl.ds(dc * LANES, LANES)
        base16 = (iota + dc * LANES) * CODE_PITCH
        for i in range(N_CAT):
            for j in range(N_CAT):
                val = emb_v[i, dsl] + emb_v[j, dsl]
                plsc.store_scatter(stT, [base16 + (i * N_CAT + j)], val)

    pltpu.sync_copy(node_hbm.at[:, pl.ds(wb, BBLK)], n_v)
    pltpu.sync_copy(t_hbm.at[:, pl.ds(wb, BBLK)], t_v)

    def emit_n(n, p):
        codes = []
        for bc in range(BBLK // LANES):
            sl = pl.ds(bc * LANES, LANES)
            nn = n_v[n, sl]
            tt = t_v[n, sl]
            s = nn + (tt * 1000.0).astype(jnp.int32)
            d1 = nn - N_CAT * ((nn * 26215) >> 17)
            d2 = s - N_CAT * ((s * 26215) >> 17)
            codes.append(jnp.where(nn == 0, N_PAIR - 1, d1 * N_CAT + d2))

        @plsc.parallel_loop(0, ENC_DIM, unroll=2)
        def _(d, codes=codes, p=p):
            dbase = d * CODE_PITCH
            for bc in range(BBLK // LANES):
                sl = pl.ds(bc * LANES, LANES)
                stage[p, d, sl] = plsc.load_gather(stT, [codes[bc] + dbase])

        pltpu.async_copy(stage.at[p], out_hbm.at[n, :, pl.ds(wb, BBLK)],
                         sems[p])

    def loop_body(i, carry):
        for p in range(2):
            @pl.when(i >= 1)
            def _(p=p):
                pltpu.make_async_copy(
                    stage.at[p], out_hbm.at[0, :, pl.ds(wb, BBLK)], sems[p]
                ).wait()
            emit_n(2 * i + p, p)
        return carry

    lax.fori_loop(0, NEIGH // 2, loop_body, 0)
    for p in range(2):
        pltpu.make_async_copy(
            stage.at[p], out_hbm.at[0, :, pl.ds(wb, BBLK)], sems[p]
        ).wait()


@functools.partial(
    pl.kernel,
    out_type=jax.ShapeDtypeStruct((NEIGH, ENC_DIM, BATCH), jnp.float32),
    mesh=plsc.VectorSubcoreMesh(core_axis_name="c", subcore_axis_name="s",
                                num_cores=NC, num_subcores=NS),
    compiler_params=pltpu.CompilerParams(use_tc_tiling_on_sc=True,
                                         needs_layout_passes=False),
    scratch_types=[
        pltpu.VMEM((N_CAT, ENC_DIM), jnp.float32),
        pltpu.VMEM((ENC_DIM * CODE_PITCH,), jnp.float32),
        pltpu.VMEM((NPAD, BBLK), jnp.int32),
        pltpu.VMEM((NPAD, BBLK), jnp.float32),
        pltpu.VMEM((2, ENC_DIM, BBLK), jnp.float32),
        pltpu.SemaphoreType.DMA,
        pltpu.SemaphoreType.DMA,
    ],
)
def _sc_encode(node_hbm, t_hbm, emb_hbm, out_hbm,
               emb_v, stT, n_v, t_v, stage, sem_out0, sem_out1):
    _sc_body(node_hbm, t_hbm, emb_hbm, out_hbm,
             emb_v, stT, n_v, t_v, stage, sem_out0, sem_out1)


def kernel(node_record, t_record, emb_table):
    nodeT = jnp.pad(node_record.transpose(1, 0), ((0, NPAD - NEIGH), (0, 0)))
    tT = jnp.pad(t_record.transpose(1, 0), ((0, NPAD - NEIGH), (0, 0)))
    outT = _sc_encode(nodeT, tT, emb_table)
    return outT.transpose(2, 0, 1)


# speedup vs baseline: 1.0949x; 1.0287x over previous
import functools

import jax
import jax.numpy as jnp
from jax import lax
from jax.experimental import pallas as pl
from jax.experimental.pallas import tpu as pltpu
from jax.experimental.pallas import tpu_sc as plsc

NUM_LAYERS = 3
ENC_DIM = 64
N_CAT = NUM_LAYERS + 2
N_PAIR = N_CAT * N_CAT
BATCH = 4096
NEIGH = 50

NC, NS, LANES = 2, 16, 16
NW = NC * NS
BBLK = BATCH // NW
CODE_PITCH = 128


def _sc_body(node_hbm, t_hbm, emb_hbm, out_hbm,
             emb_v, stT, n_v, t_v, stage, sem_out0, sem_out1):
    cid = lax.axis_index("c")
    sid = lax.axis_index("s")
    wid = sid * NC + cid
    wb = pl.multiple_of(wid * BBLK, BBLK)
    sems = (sem_out0, sem_out1)

    pltpu.sync_copy(emb_hbm, emb_v)
    iota = lax.iota(jnp.int32, LANES)
    for dc in range(ENC_DIM // LANES):
        dsl = pl.ds(dc * LANES, LANES)
        base16 = (iota + dc * LANES) * CODE_PITCH
        for i in range(N_CAT):
            for j in range(N_CAT):
                val = emb_v[i, dsl] + emb_v[j, dsl]
                plsc.store_scatter(stT, [base16 + (i * N_CAT + j)], val)

    pltpu.sync_copy(node_hbm.at[:, pl.ds(wb, BBLK)], n_v)
    pltpu.sync_copy(t_hbm.at[:, pl.ds(wb, BBLK)], t_v)

    def emit_n(n, p):
        codes = []
        for bc in range(BBLK // LANES):
            sl = pl.ds(bc * LANES, LANES)
            nn = n_v[n, sl]
            tt = t_v[n, sl]
            s = nn + (tt * 1000.0).astype(jnp.int32)
            d1 = nn - N_CAT * ((nn * 26215) >> 17)
            d2 = s - N_CAT * ((s * 26215) >> 17)
            codes.append(jnp.where(nn == 0, N_PAIR - 1, d1 * N_CAT + d2))

        @plsc.parallel_loop(0, ENC_DIM, unroll=2)
        def _(d, codes=codes, p=p):
            dbase = d * CODE_PITCH
            for bc in range(BBLK // LANES):
                sl = pl.ds(bc * LANES, LANES)
                stage[p, d, sl] = plsc.load_gather(stT, [codes[bc] + dbase])

        pltpu.async_copy(stage.at[p], out_hbm.at[n, :, pl.ds(wb, BBLK)],
                         sems[p])

    def loop_body(i, carry):
        for p in range(2):
            @pl.when(i >= 1)
            def _(p=p):
                pltpu.make_async_copy(
                    stage.at[p], out_hbm.at[0, :, pl.ds(wb, BBLK)], sems[p]
                ).wait()
            emit_n(2 * i + p, p)
        return carry

    lax.fori_loop(0, NEIGH // 2, loop_body, 0)
    for p in range(2):
        pltpu.make_async_copy(
            stage.at[p], out_hbm.at[0, :, pl.ds(wb, BBLK)], sems[p]
        ).wait()


@functools.partial(
    pl.kernel,
    out_type=jax.ShapeDtypeStruct((NEIGH, ENC_DIM, BATCH), jnp.float32),
    mesh=plsc.VectorSubcoreMesh(core_axis_name="c", subcore_axis_name="s",
                                num_cores=NC, num_subcores=NS),
    compiler_params=pltpu.CompilerParams(use_tc_tiling_on_sc=True,
                                         needs_layout_passes=False),
    scratch_types=[
        pltpu.VMEM((N_CAT, ENC_DIM), jnp.float32),
        pltpu.VMEM((ENC_DIM * CODE_PITCH,), jnp.float32),
        pltpu.VMEM((NEIGH, BBLK), jnp.int32),
        pltpu.VMEM((NEIGH, BBLK), jnp.float32),
        pltpu.VMEM((2, ENC_DIM, BBLK), jnp.float32),
        pltpu.SemaphoreType.DMA,
        pltpu.SemaphoreType.DMA,
    ],
)
def _sc_encode(node_hbm, t_hbm, emb_hbm, out_hbm,
               emb_v, stT, n_v, t_v, stage, sem_out0, sem_out1):
    _sc_body(node_hbm, t_hbm, emb_hbm, out_hbm,
             emb_v, stT, n_v, t_v, stage, sem_out0, sem_out1)


def kernel(node_record, t_record, emb_table):
    nodeT = node_record.transpose(1, 0)
    tT = t_record.transpose(1, 0)
    outT = _sc_encode(nodeT, tT, emb_table)
    return outT.transpose(2, 0, 1)
